# baseline (speedup 1.0000x reference)
import jax
import jax.numpy as jnp
from jax.experimental import pallas as pl
from jax.experimental.pallas import tpu as pltpu

_B, _F, _NC, _P = 8, 256, 1024, 576


def _tc_body(x_ref, c_ref, y_ref, loss_ref):
    b = pl.program_id(0)
    nb = pl.num_programs(0)
    xb = x_ref[0]
    C = c_ref[...]
    c_sq = jnp.sum(C * C, axis=0, keepdims=True)
    S2 = jax.lax.dot_general(C, xb, (((0,), (0,)), ((), ())),
                             preferred_element_type=jnp.float32)
    score = c_sq.T - 2.0 * S2
    m = jnp.min(score, axis=0, keepdims=True)
    iota0 = jax.lax.broadcasted_iota(jnp.int32, (_NC, _P), 0)
    idx = jnp.min(jnp.where(score == m, iota0, _NC), axis=0)
    oh = (iota0 == idx[None, :]).astype(jnp.float32)
    xq = jax.lax.dot_general(C, oh, (((1,), (0,)), ((), ())),
                             preferred_element_type=jnp.float32,
                             precision=jax.lax.Precision.HIGHEST)
    y_ref[0] = xq
    x_sq = jnp.sum(xb * xb, axis=0)
    partial = jnp.sum(x_sq + m[0])

    @pl.when(b == 0)
    def _():
        loss_ref[0, 0] = 0.0

    loss_ref[0, 0] += partial

    @pl.when(b == nb - 1)
    def _():
        loss_ref[0, 0] = loss_ref[0, 0] / (_B * _F * _P)


def kernel(x, centroids):
    x3 = x.reshape(_B, _F, _P)
    y, loss = pl.pallas_call(
        _tc_body,
        grid=(_B,),
        in_specs=[
            pl.BlockSpec((1, _F, _P), lambda b: (b, 0, 0)),
            pl.BlockSpec((_F, _NC), lambda b: (0, 0)),
        ],
        out_specs=[
            pl.BlockSpec((1, _F, _P), lambda b: (b, 0, 0)),
            pl.BlockSpec(memory_space=pltpu.SMEM, block_shape=(1, 1),
                         index_map=lambda b: (0, 0)),
        ],
        out_shape=[
            jax.ShapeDtypeStruct((_B, _F, _P), jnp.float32),
            jax.ShapeDtypeStruct((1, 1), jnp.float32),
        ],
        compiler_params=pltpu.CompilerParams(
            dimension_semantics=("arbitrary",),
        ),
    )(x3, centroids)
    return y.reshape(_B, _F, 24, 24), loss[0, 0]
